# baseline (speedup 1.0000x reference)
import functools

import jax
import jax.numpy as jnp
from jax import lax
from jax.experimental import pallas as pl
from jax.experimental.pallas import tpu as pltpu
from jax.experimental.pallas import tpu_sc as plsc

S = 7
NCELL = S * S
D = 30
ROW = NCELL * D
BATCH = 64
OUTW = 512
CLS_OFF = 320
KEEP_OFF = 384
CONF_THRES = 0.5
NMS_THRES = 0.7
GRID = 64.0
WIMG = 448.0
NEG_INF = float("-inf")

_mesh = plsc.VectorSubcoreMesh(core_axis_name="c", subcore_axis_name="s")


@functools.partial(
    pl.kernel,
    out_type=jax.ShapeDtypeStruct((BATCH, OUTW), jnp.float32),
    mesh=_mesh,
    compiler_params=pltpu.CompilerParams(needs_layout_passes=False),
    scratch_types=[
        pltpu.VMEM((2, ROW), jnp.float32),
        pltpu.VMEM((128,), jnp.float32),
        pltpu.VMEM((128,), jnp.float32),
        pltpu.VMEM((128,), jnp.float32),
        pltpu.VMEM((128,), jnp.float32),
        pltpu.VMEM((128,), jnp.float32),
        pltpu.VMEM((128,), jnp.float32),
        pltpu.VMEM((128,), jnp.float32),
        pltpu.VMEM((128,), jnp.float32),
        pltpu.VMEM((128,), jnp.float32),
        pltpu.VMEM((128,), jnp.float32),
        pltpu.VMEM((128,), jnp.float32),
        pltpu.VMEM((128,), jnp.float32),
        pltpu.VMEM((2, OUTW), jnp.float32),
        pltpu.SemaphoreType.DMA,
        pltpu.SemaphoreType.DMA,
    ],
)
def _yolo_sc(outp_hbm, out_hbm,
             buf, x1u, y1u, x2u, y2u, cfa, cpa, sma,
             x1o, y1o, x2o, y2o, ara, db,
             sem_in, sem_out):
    wid = lax.axis_index("s") * 2 + lax.axis_index("c")
    img_a = wid * 2
    lane = jnp.arange(16, dtype=jnp.int32)
    zeros16 = jnp.zeros((16,), jnp.int32)

    pltpu.async_copy(outp_hbm.at[pl.ds(img_a, 2)], buf, sem_in).wait()

    def decode_body(i, _):
        k = i // 4
        cb = (i % 4) * 16
        g = lane + cb
        gc = jnp.minimum(g, NCELL - 1)
        m49 = g < NCELL
        kf = jnp.full((16,), k, jnp.int32)
        base = gc * D

        def ld(f):
            return plsc.load_gather(buf, [kf, base + f])

        conf0 = ld(4)
        conf1 = ld(9)
        use1 = conf1 > conf0
        conf = jnp.maximum(conf0, conf1)
        boff = base + jnp.where(use1, 5, 0)
        bx = plsc.load_gather(buf, [kf, boff])
        by = plsc.load_gather(buf, [kf, boff + 1])
        bw = plsc.load_gather(buf, [kf, boff + 2])
        bh = plsc.load_gather(buf, [kf, boff + 3])
        colf = (gc % S).astype(jnp.float32)
        rowf = (gc // S).astype(jnp.float32)
        cx = (bx + colf) * GRID
        cy = (by + rowf) * GRID
        w = bw * WIMG
        h = bh * WIMG
        x1 = cx - w * 0.5
        y1 = cy - h * 0.5
        x2 = cx + w * 0.5
        y2 = cy + h * 0.5
        best = ld(10)
        bidx = zeros16
        for kk in range(1, 20):
            v = ld(10 + kk)
            bidx = jnp.where(v > best, kk, bidx)
            best = jnp.maximum(best, v)
        valid = (conf > CONF_THRES) & m49
        offv = bidx.astype(jnp.float32) * (2.0 * WIMG + 1.0)
        xo1 = x1 + offv
        xo2 = x2 + offv
        yo1 = y1 + offv
        yo2 = y2 + offv
        area = jnp.maximum(xo2 - xo1, 0.0) * jnp.maximum(yo2 - yo1, 0.0)
        sl = pl.ds(i * 16, 16)
        x1u[sl] = x1
        y1u[sl] = y1
        x2u[sl] = x2
        y2u[sl] = y2
        cfa[sl] = conf
        cpa[sl] = best
        sma[sl] = jnp.where(valid, conf, NEG_INF)
        x1o[sl] = xo1
        y1o[sl] = yo1
        x2o[sl] = xo2
        y2o[sl] = yo2
        ara[sl] = area
        db[k, pl.ds(CLS_OFF + cb, 16)] = bidx.astype(jnp.float32)
        return 0

    lax.fori_loop(0, 8, decode_body, 0)

    st0 = tuple(sma[pl.ds(i * 16, 16)] for i in range(8))
    kp0 = tuple(zeros16 for _ in range(8))

    def nms_body(_, carry):
        out_st, out_kp = [], []
        for k in range(2):
            koff = k * 64
            st = carry[0][k * 4:k * 4 + 4]
            kp = carry[1][k * 4:k * 4 + 4]
            s0, s1, s2, s3 = st
            mx = jnp.max(jnp.maximum(jnp.maximum(s0, s1), jnp.maximum(s2, s3)))
            cands = [
                plsc.all_reduce_ffs(s_c == mx) + c * 16
                for c, s_c in enumerate(st)
            ]
            cands = [
                jnp.where(cand >= (c + 1) * 16, 999, cand)
                for c, cand in enumerate(cands)
            ]
            jsv = jnp.minimum(jnp.minimum(cands[0], cands[1]),
                              jnp.minimum(cands[2], cands[3]))
            picked = mx != NEG_INF
            jv = jsv + koff
            x1c = plsc.load_gather(x1o, [jv])
            y1c = plsc.load_gather(y1o, [jv])
            x2c = plsc.load_gather(x2o, [jv])
            y2c = plsc.load_gather(y2o, [jv])
            arc = plsc.load_gather(ara, [jv])
            for c, s_c in enumerate(st):
                idxs = lane + c * 16
                live = s_c != NEG_INF
                sl = pl.ds(koff + c * 16, 16)
                xx1 = jnp.maximum(x1o[sl], x1c)
                yy1 = jnp.maximum(y1o[sl], y1c)
                xx2 = jnp.minimum(x2o[sl], x2c)
                yy2 = jnp.minimum(y2o[sl], y2c)
                inter = (jnp.maximum(xx2 - xx1, 0.0)
                         * jnp.maximum(yy2 - yy1, 0.0))
                union = ara[sl] + arc - inter
                iou = inter / jnp.maximum(union, 1e-9)
                sup = (iou > NMS_THRES) & live
                out_st.append(jnp.where(sup | (idxs == jsv), NEG_INF, s_c))
                out_kp.append(jnp.where((idxs == jsv) & picked, 1, kp[c]))
        return tuple(out_st), tuple(out_kp)

    _, kp_fin = lax.fori_loop(0, NCELL, nms_body, (st0, kp0))

    for i in range(8):
        k = i // 4
        cb = (i % 4) * 16
        g = lane + cb
        gc = jnp.minimum(g, NCELL - 1)
        m49 = g < NCELL
        sl = pl.ds(i * 16, 16)
        kv = kp_fin[i] != 0
        kvec = jnp.full((16,), k, jnp.int32)
        for f, arr in enumerate((x1u, y1u, x2u, y2u, cfa, cpa)):
            plsc.store_scatter(db, [kvec, gc * 6 + f],
                               jnp.where(kv, arr[sl], 0.0), mask=m49)
        db[k, pl.ds(KEEP_OFF + cb, 16)] = kp_fin[i].astype(jnp.float32)
    pltpu.async_copy(db, out_hbm.at[pl.ds(img_a, 2)], sem_out).wait()


def kernel(images, outputs, prefix=0):
    del images, prefix
    packed = _yolo_sc(outputs)
    det = packed[:, : NCELL * 6].reshape(BATCH, NCELL, 6)
    cls_idx = packed[:, CLS_OFF:CLS_OFF + NCELL].astype(jnp.int32)
    keep = packed[:, KEEP_OFF:KEEP_OFF + NCELL] != 0.0
    return det, cls_idx, keep


# speedup vs baseline: 1.0582x; 1.0582x over previous
import functools

import jax
import jax.numpy as jnp
from jax import lax
from jax.experimental import pallas as pl
from jax.experimental.pallas import tpu as pltpu
from jax.experimental.pallas import tpu_sc as plsc

S = 7
NCELL = S * S
D = 30
ROW = NCELL * D
BATCH = 64
NPAD = 64
DET_PAD = 320
CONF_THRES = 0.5
NMS_THRES = 0.7
GRID = 64.0
WIMG = 448.0
NEG_INF = float("-inf")

_mesh = plsc.VectorSubcoreMesh(core_axis_name="c", subcore_axis_name="s")


@functools.partial(
    pl.kernel,
    out_type=(
        jax.ShapeDtypeStruct((BATCH, DET_PAD), jnp.float32),
        jax.ShapeDtypeStruct((BATCH, 2 * NPAD), jnp.int32),
    ),
    mesh=_mesh,
    compiler_params=pltpu.CompilerParams(needs_layout_passes=False),
    scratch_types=[
        pltpu.VMEM((2, ROW), jnp.float32),
        pltpu.VMEM((128,), jnp.float32),
        pltpu.VMEM((128,), jnp.float32),
        pltpu.VMEM((128,), jnp.float32),
        pltpu.VMEM((128,), jnp.float32),
        pltpu.VMEM((128,), jnp.float32),
        pltpu.VMEM((128,), jnp.float32),
        pltpu.VMEM((128,), jnp.float32),
        pltpu.VMEM((128,), jnp.float32),
        pltpu.VMEM((128,), jnp.float32),
        pltpu.VMEM((128,), jnp.float32),
        pltpu.VMEM((128,), jnp.float32),
        pltpu.VMEM((128,), jnp.float32),
        pltpu.VMEM((2, 2 * NPAD), jnp.int32),
        pltpu.VMEM((2, DET_PAD), jnp.float32),
        pltpu.SemaphoreType.DMA,
        pltpu.SemaphoreType.DMA,
        pltpu.SemaphoreType.DMA,
    ],
)
def _yolo_sc(outp_hbm, det_hbm, misc_hbm,
             buf, x1u, y1u, x2u, y2u, cfa, cpa, sma,
             x1o, y1o, x2o, y2o, ara, misc, db,
             sem_in, sem_d, sem_m):
    wid = lax.axis_index("s") * 2 + lax.axis_index("c")
    img_a = wid * 2
    lane = jnp.arange(16, dtype=jnp.int32)
    zeros16 = jnp.zeros((16,), jnp.int32)

    pltpu.async_copy(outp_hbm.at[pl.ds(img_a, 2)], buf, sem_in).wait()

    def decode_body(i, _):
        k = i // 4
        cb = (i % 4) * 16
        g = lane + cb
        gc = jnp.minimum(g, NCELL - 1)
        m49 = g < NCELL
        kf = jnp.full((16,), k, jnp.int32)
        base = gc * D

        def ld(f):
            return plsc.load_gather(buf, [kf, base + f])

        conf0 = ld(4)
        conf1 = ld(9)
        use1 = conf1 > conf0
        conf = jnp.maximum(conf0, conf1)
        boff = base + jnp.where(use1, 5, 0)
        bx = plsc.load_gather(buf, [kf, boff])
        by = plsc.load_gather(buf, [kf, boff + 1])
        bw = plsc.load_gather(buf, [kf, boff + 2])
        bh = plsc.load_gather(buf, [kf, boff + 3])
        colf = (gc % S).astype(jnp.float32)
        rowf = (gc // S).astype(jnp.float32)
        cx = (bx + colf) * GRID
        cy = (by + rowf) * GRID
        w = bw * WIMG
        h = bh * WIMG
        x1 = cx - w * 0.5
        y1 = cy - h * 0.5
        x2 = cx + w * 0.5
        y2 = cy + h * 0.5
        best = ld(10)
        bidx = zeros16
        for kk in range(1, 20):
            v = ld(10 + kk)
            bidx = jnp.where(v > best, kk, bidx)
            best = jnp.maximum(best, v)
        valid = (conf > CONF_THRES) & m49
        offv = bidx.astype(jnp.float32) * (2.0 * WIMG + 1.0)
        xo1 = x1 + offv
        xo2 = x2 + offv
        yo1 = y1 + offv
        yo2 = y2 + offv
        area = jnp.maximum(xo2 - xo1, 0.0) * jnp.maximum(yo2 - yo1, 0.0)
        sl = pl.ds(i * 16, 16)
        x1u[sl] = x1
        y1u[sl] = y1
        x2u[sl] = x2
        y2u[sl] = y2
        cfa[sl] = conf
        cpa[sl] = best
        sma[sl] = jnp.where(valid, conf, NEG_INF)
        x1o[sl] = xo1
        y1o[sl] = yo1
        x2o[sl] = xo2
        y2o[sl] = yo2
        ara[sl] = area
        cb16 = (i % 4) * 16
        misc[k, pl.ds(cb16, 16)] = bidx
        return 0

    lax.fori_loop(0, 8, decode_body, 0)

    st0 = tuple(sma[pl.ds(i * 16, 16)] for i in range(8))
    kp0 = tuple(zeros16 for _ in range(8))

    def nms_body(_, carry):
        out_st, out_kp = [], []
        for k in range(2):
            koff = k * 64
            st = carry[0][k * 4:k * 4 + 4]
            kp = carry[1][k * 4:k * 4 + 4]
            s0, s1, s2, s3 = st
            mx = jnp.max(jnp.maximum(jnp.maximum(s0, s1), jnp.maximum(s2, s3)))
            cands = [
                plsc.all_reduce_ffs(s_c == mx) + c * 16
                for c, s_c in enumerate(st)
            ]
            cands = [
                jnp.where(cand >= (c + 1) * 16, 999, cand)
                for c, cand in enumerate(cands)
            ]
            jsv = jnp.minimum(jnp.minimum(cands[0], cands[1]),
                              jnp.minimum(cands[2], cands[3]))
            picked = mx != NEG_INF
            jv = jsv + koff
            x1c = plsc.load_gather(x1o, [jv])
            y1c = plsc.load_gather(y1o, [jv])
            x2c = plsc.load_gather(x2o, [jv])
            y2c = plsc.load_gather(y2o, [jv])
            arc = plsc.load_gather(ara, [jv])
            for c, s_c in enumerate(st):
                idxs = lane + c * 16
                live = s_c != NEG_INF
                sl = pl.ds(koff + c * 16, 16)
                xx1 = jnp.maximum(x1o[sl], x1c)
                yy1 = jnp.maximum(y1o[sl], y1c)
                xx2 = jnp.minimum(x2o[sl], x2c)
                yy2 = jnp.minimum(y2o[sl], y2c)
                inter = (jnp.maximum(xx2 - xx1, 0.0)
                         * jnp.maximum(yy2 - yy1, 0.0))
                union = ara[sl] + arc - inter
                iou = inter / jnp.maximum(union, 1e-9)
                sup = (iou > NMS_THRES) & live
                out_st.append(jnp.where(sup | (idxs == jsv), NEG_INF, s_c))
                out_kp.append(jnp.where((idxs == jsv) & picked, 1, kp[c]))
        return tuple(out_st), tuple(out_kp)

    _, kp_fin = lax.fori_loop(0, NCELL, nms_body, (st0, kp0))

    for i in range(8):
        k = i // 4
        cb = (i % 4) * 16
        g = lane + cb
        gc = jnp.minimum(g, NCELL - 1)
        m49 = g < NCELL
        sl = pl.ds(i * 16, 16)
        kv = kp_fin[i] != 0
        kvec = jnp.full((16,), k, jnp.int32)
        for f, arr in enumerate((x1u, y1u, x2u, y2u, cfa, cpa)):
            plsc.store_scatter(db, [kvec, gc * 6 + f],
                               jnp.where(kv, arr[sl], 0.0), mask=m49)
        misc[k, pl.ds(NPAD + cb, 16)] = kp_fin[i]
    pltpu.async_copy(db, det_hbm.at[pl.ds(img_a, 2)], sem_d).wait()
    pltpu.async_copy(misc, misc_hbm.at[pl.ds(img_a, 2)], sem_m).wait()


def kernel(images, outputs, prefix=0):
    del images, prefix
    det_p, misc_p = _yolo_sc(outputs)
    det = det_p[:, : NCELL * 6].reshape(BATCH, NCELL, 6)
    return det, misc_p[:, :NCELL], misc_p[:, NPAD:NPAD + NCELL] != 0


# speedup vs baseline: 1.0714x; 1.0125x over previous
import functools

import jax
import jax.numpy as jnp
from jax import lax
from jax.experimental import pallas as pl
from jax.experimental.pallas import tpu as pltpu
from jax.experimental.pallas import tpu_sc as plsc

S = 7
NCELL = S * S
D = 30
ROW = NCELL * D
BATCH = 64
NPAD = 64
DET_PAD = 320
CONF_THRES = 0.5
NMS_THRES = 0.7
GRID = 64.0
WIMG = 448.0
NEG_INF = float("-inf")

_mesh = plsc.VectorSubcoreMesh(core_axis_name="c", subcore_axis_name="s")


@functools.partial(
    pl.kernel,
    out_type=(
        jax.ShapeDtypeStruct((BATCH, DET_PAD), jnp.float32),
        jax.ShapeDtypeStruct((BATCH, 2 * NPAD), jnp.int32),
    ),
    mesh=_mesh,
    compiler_params=pltpu.CompilerParams(needs_layout_passes=False),
    scratch_types=[
        pltpu.VMEM((2, ROW), jnp.float32),
        pltpu.VMEM((128,), jnp.float32),
        pltpu.VMEM((128,), jnp.float32),
        pltpu.VMEM((128,), jnp.float32),
        pltpu.VMEM((128,), jnp.float32),
        pltpu.VMEM((128,), jnp.float32),
        pltpu.VMEM((128,), jnp.float32),
        pltpu.VMEM((128,), jnp.float32),
        pltpu.VMEM((128,), jnp.float32),
        pltpu.VMEM((128,), jnp.float32),
        pltpu.VMEM((128,), jnp.float32),
        pltpu.VMEM((128,), jnp.float32),
        pltpu.VMEM((128,), jnp.float32),
        pltpu.VMEM((2, 2 * NPAD), jnp.int32),
        pltpu.VMEM((2, DET_PAD), jnp.float32),
        pltpu.SemaphoreType.DMA,
        pltpu.SemaphoreType.DMA,
        pltpu.SemaphoreType.DMA,
    ],
)
def _yolo_sc(outp_hbm, det_hbm, misc_hbm,
             buf, x1u, y1u, x2u, y2u, cfa, cpa, sma,
             x1o, y1o, x2o, y2o, ara, misc, db,
             sem_in, sem_d, sem_m):
    wid = lax.axis_index("s") * 2 + lax.axis_index("c")
    img_a = wid * 2
    lane = jnp.arange(16, dtype=jnp.int32)
    zeros16 = jnp.zeros((16,), jnp.int32)

    pltpu.async_copy(outp_hbm.at[pl.ds(img_a, 2)], buf, sem_in).wait()

    def decode_body(i, _):
        k = i // 4
        cb = (i % 4) * 16
        g = lane + cb
        gc = jnp.minimum(g, NCELL - 1)
        m49 = g < NCELL
        kf = jnp.full((16,), k, jnp.int32)
        base = gc * D

        def ld(f):
            return plsc.load_gather(buf, [kf, base + f])

        conf0 = ld(4)
        conf1 = ld(9)
        use1 = conf1 > conf0
        conf = jnp.maximum(conf0, conf1)
        boff = base + jnp.where(use1, 5, 0)
        bx = plsc.load_gather(buf, [kf, boff])
        by = plsc.load_gather(buf, [kf, boff + 1])
        bw = plsc.load_gather(buf, [kf, boff + 2])
        bh = plsc.load_gather(buf, [kf, boff + 3])
        colf = (gc % S).astype(jnp.float32)
        rowf = (gc // S).astype(jnp.float32)
        cx = (bx + colf) * GRID
        cy = (by + rowf) * GRID
        w = bw * WIMG
        h = bh * WIMG
        x1 = cx - w * 0.5
        y1 = cy - h * 0.5
        x2 = cx + w * 0.5
        y2 = cy + h * 0.5
        best = ld(10)
        bidx = zeros16
        for kk in range(1, 20):
            v = ld(10 + kk)
            bidx = jnp.where(v > best, kk, bidx)
            best = jnp.maximum(best, v)
        valid = (conf > CONF_THRES) & m49
        offv = bidx.astype(jnp.float32) * (2.0 * WIMG + 1.0)
        xo1 = x1 + offv
        xo2 = x2 + offv
        yo1 = y1 + offv
        yo2 = y2 + offv
        area = jnp.maximum(xo2 - xo1, 0.0) * jnp.maximum(yo2 - yo1, 0.0)
        sl = pl.ds(i * 16, 16)
        x1u[sl] = x1
        y1u[sl] = y1
        x2u[sl] = x2
        y2u[sl] = y2
        cfa[sl] = conf
        cpa[sl] = best
        sma[sl] = jnp.where(valid, conf, NEG_INF)
        x1o[sl] = xo1
        y1o[sl] = yo1
        x2o[sl] = xo2
        y2o[sl] = yo2
        ara[sl] = area
        cb16 = (i % 4) * 16
        misc[k, pl.ds(cb16, 16)] = bidx
        return 0

    lax.fori_loop(0, 8, decode_body, 0)

    st0 = tuple(sma[pl.ds(i * 16, 16)] for i in range(8))
    kp0 = tuple(zeros16 for _ in range(8))

    def nms_body(carry):
        out_st, out_kp = [], []
        pickeds = []
        for k in range(2):
            koff = k * 64
            st = carry[0][k * 4:k * 4 + 4]
            kp = carry[1][k * 4:k * 4 + 4]
            s0, s1, s2, s3 = st
            mx = jnp.max(jnp.maximum(jnp.maximum(s0, s1), jnp.maximum(s2, s3)))
            cands = [
                plsc.all_reduce_ffs(s_c == mx) + c * 16
                for c, s_c in enumerate(st)
            ]
            cands = [
                jnp.where(cand >= (c + 1) * 16, 999, cand)
                for c, cand in enumerate(cands)
            ]
            jsv = jnp.minimum(jnp.minimum(cands[0], cands[1]),
                              jnp.minimum(cands[2], cands[3]))
            picked = mx != NEG_INF
            pickeds.append(picked)
            jv = jsv + koff
            x1c = plsc.load_gather(x1o, [jv])
            y1c = plsc.load_gather(y1o, [jv])
            x2c = plsc.load_gather(x2o, [jv])
            y2c = plsc.load_gather(y2o, [jv])
            arc = plsc.load_gather(ara, [jv])
            for c, s_c in enumerate(st):
                idxs = lane + c * 16
                live = s_c != NEG_INF
                sl = pl.ds(koff + c * 16, 16)
                xx1 = jnp.maximum(x1o[sl], x1c)
                yy1 = jnp.maximum(y1o[sl], y1c)
                xx2 = jnp.minimum(x2o[sl], x2c)
                yy2 = jnp.minimum(y2o[sl], y2c)
                inter = (jnp.maximum(xx2 - xx1, 0.0)
                         * jnp.maximum(yy2 - yy1, 0.0))
                union = ara[sl] + arc - inter
                iou = inter / jnp.maximum(union, 1e-9)
                sup = (iou > NMS_THRES) & live
                out_st.append(jnp.where(sup | (idxs == jsv), NEG_INF, s_c))
                out_kp.append(jnp.where((idxs == jsv) & picked, 1, kp[c]))
        return tuple(out_st), tuple(out_kp), pickeds[0] | pickeds[1]

    _, kp_fin, _ = lax.while_loop(lambda c: c[2], nms_body,
                                  (st0, kp0, jnp.bool_(True)))

    for i in range(8):
        k = i // 4
        cb = (i % 4) * 16
        g = lane + cb
        gc = jnp.minimum(g, NCELL - 1)
        m49 = g < NCELL
        sl = pl.ds(i * 16, 16)
        kv = kp_fin[i] != 0
        kvec = jnp.full((16,), k, jnp.int32)
        for f, arr in enumerate((x1u, y1u, x2u, y2u, cfa, cpa)):
            plsc.store_scatter(db, [kvec, gc * 6 + f],
                               jnp.where(kv, arr[sl], 0.0), mask=m49)
        misc[k, pl.ds(NPAD + cb, 16)] = kp_fin[i]
    pltpu.async_copy(db, det_hbm.at[pl.ds(img_a, 2)], sem_d).wait()
    pltpu.async_copy(misc, misc_hbm.at[pl.ds(img_a, 2)], sem_m).wait()


def kernel(images, outputs, prefix=0):
    del images, prefix
    det_p, misc_p = _yolo_sc(outputs)
    det = det_p[:, : NCELL * 6].reshape(BATCH, NCELL, 6)
    return det, misc_p[:, :NCELL], misc_p[:, NPAD:NPAD + NCELL] != 0
